# Initial kernel scaffold; baseline (speedup 1.0000x reference)
import jax
import jax.numpy as jnp
from jax.experimental import pallas as pl


def kernel(feat, edge_index, etypes, truth_value, weight, h_bias):
    raise NotImplementedError("write your pallas kernel here")


# speedup vs baseline: 8.8652x; 8.8652x over previous
import functools

import jax
import jax.numpy as jnp
from jax import lax
from jax.experimental import pallas as pl
from jax.experimental.pallas import tpu as pltpu
from jax.experimental.pallas import tpu_sc as plsc

N = 25000
E = 100000
IN_FEAT = 16
OUT_FEAT = 16
NUM_RELS = 16
NUM_RULES = 4

NW = 32
EW = 3136
E_PAD = NW * EW
SUB = 112
NSUB = EW // SUB
GRP = SUB // 16
N_PAD = 25088
ROWS_PER_TILE = N_PAD // 16
XROW = NUM_RULES * OUT_FEAT


def _xform_body(f_ref, w_ref, b_ref, o_ref):
    o_ref[...] = (
        jnp.dot(f_ref[...], w_ref[...], preferred_element_type=jnp.float32)
        + b_ref[...]
    )


def _compute_x(feat, w2, b2):
    blk = 1000
    grid = N // blk
    return pl.pallas_call(
        _xform_body,
        grid=(grid,),
        in_specs=[
            pl.BlockSpec((blk, IN_FEAT), lambda i: (i, 0)),
            pl.BlockSpec((IN_FEAT, NUM_RELS * XROW), lambda i: (0, 0)),
            pl.BlockSpec((1, NUM_RELS * XROW), lambda i: (0, 0)),
        ],
        out_specs=pl.BlockSpec((blk, NUM_RELS * XROW), lambda i: (i, 0)),
        out_shape=jax.ShapeDtypeStruct((N, NUM_RELS * XROW), jnp.float32),
    )(feat, w2, b2)


def _edge_body(xrows, src_w, et_w, dst_w, tv_w, out_partial,
               src_v, et_v, dst_v, tv_v, xidx_v, xbuf, msg_v, accum, sem):
    cid = lax.axis_index("c")
    sid = lax.axis_index("s")
    wid = cid * 16 + sid

    iota = lax.iota(jnp.int32, 16)
    zeros16 = jnp.zeros((16,), jnp.float32)

    pltpu.sync_copy(src_w.at[wid], src_v)
    pltpu.sync_copy(et_w.at[wid], et_v)
    pltpu.sync_copy(dst_w.at[wid], dst_v)
    pltpu.sync_copy(tv_w.at[wid], tv_v.at[pl.ds(0, EW * NUM_RULES)])

    for i in range(SUB):
        msg_v[i, :] = zeros16
    for q in range(ROWS_PER_TILE // SUB):
        pltpu.sync_copy(msg_v, accum.at[pl.ds(sid * ROWS_PER_TILE + q * SUB, SUB)])

    for c in range(NSUB):
        for g in range(GRP):
            s16 = src_v[pl.ds(c * SUB + g * 16, 16)]
            e16 = et_v[pl.ds(c * SUB + g * 16, 16)]
            xidx_v[c, pl.ds(g * 16, 16)] = s16 * NUM_RELS + e16

    plsc.subcore_barrier()

    def sub_chunk(c, _):
        pltpu.async_copy(xrows.at[xidx_v.at[c]], xbuf, sem).wait()
        base4 = c * (SUB * NUM_RULES)
        for i in range(SUB):
            tvv = tv_v[pl.ds(base4 + i * NUM_RULES, 16)]
            acc = tvv[0] * xbuf[i, pl.ds(0, 16)]
            for r in range(1, NUM_RULES):
                acc = acc + tvv[r] * xbuf[i, pl.ds(r * 16, 16)]
            msg_v[i, :] = acc
        pltpu.sync_copy(msg_v, accum.at[dst_v.at[c]], add=True)
        return ()

    lax.fori_loop(0, NSUB, sub_chunk, (), unroll=False)

    plsc.subcore_barrier()

    pltpu.sync_copy(
        accum.at[pl.ds(sid * ROWS_PER_TILE, ROWS_PER_TILE)],
        out_partial.at[cid, pl.ds(sid * ROWS_PER_TILE, ROWS_PER_TILE)],
    )


def _edge_pass(xrows, src_w, et_w, dst_w, tv_w):
    mesh = plsc.VectorSubcoreMesh(core_axis_name="c", subcore_axis_name="s")
    fn = pl.kernel(
        _edge_body,
        mesh=mesh,
        compiler_params=pltpu.CompilerParams(use_tc_tiling_on_sc=False),
        out_type=jax.ShapeDtypeStruct((2, N_PAD, OUT_FEAT), jnp.float32),
        scratch_types=[
            pltpu.VMEM((EW,), jnp.int32),
            pltpu.VMEM((EW,), jnp.int32),
            pltpu.VMEM((NSUB, SUB), jnp.int32),
            pltpu.VMEM((EW * NUM_RULES + 16,), jnp.float32),
            pltpu.VMEM((NSUB, SUB), jnp.int32),
            pltpu.VMEM((SUB, XROW), jnp.float32),
            pltpu.VMEM((SUB, OUT_FEAT), jnp.float32),
            pltpu.VMEM_SHARED((N_PAD, OUT_FEAT), jnp.float32),
            pltpu.SemaphoreType.DMA,
        ],
    )
    return fn(xrows, src_w, et_w, dst_w, tv_w)


def _sum_body(p_ref, o_ref):
    o_ref[...] = p_ref[0] + p_ref[1]


def _sum_partials(partial):
    return pl.pallas_call(
        _sum_body,
        grid=(16,),
        in_specs=[pl.BlockSpec((2, ROWS_PER_TILE, OUT_FEAT), lambda i: (0, i, 0))],
        out_specs=pl.BlockSpec((ROWS_PER_TILE, OUT_FEAT), lambda i: (i, 0)),
        out_shape=jax.ShapeDtypeStruct((N_PAD, OUT_FEAT), jnp.float32),
    )(partial)


@jax.jit
def kernel(feat, edge_index, etypes, truth_value, weight, h_bias):
    w2 = weight.transpose(2, 0, 1, 3).reshape(IN_FEAT, NUM_RELS * XROW)
    b2 = h_bias.reshape(1, NUM_RELS * XROW)

    x2d = _compute_x(feat, w2, b2)
    xrows = x2d.reshape(N * NUM_RELS, XROW)

    src = edge_index[0]
    dst = edge_index[1]
    pad = E_PAD - E
    src_p = jnp.concatenate([src, jnp.zeros((pad,), jnp.int32)])
    dst_p = jnp.concatenate([dst, jnp.zeros((pad,), jnp.int32)])
    et_p = jnp.concatenate([etypes, jnp.zeros((pad,), jnp.int32)])
    tv_p = jnp.concatenate(
        [truth_value.reshape(E, NUM_RULES),
         jnp.zeros((pad, NUM_RULES), jnp.float32)])

    src_w = src_p.reshape(NW, EW)
    et_w = et_p.reshape(NW, EW)
    dst_w = dst_p.reshape(NW, NSUB, SUB)
    tv_w = tv_p.reshape(NW, EW * NUM_RULES)

    partial = _edge_pass(xrows, src_w, et_w, dst_w, tv_w)
    summed = _sum_partials(partial)
    return summed[:N].reshape(N, 1, OUT_FEAT)
